# baseline (speedup 1.0000x reference)
import numpy as np

import jax
import jax.numpy as jnp
from jax import lax
from jax.experimental import pallas as pl
from jax.experimental.pallas import tpu as pltpu
from jax.experimental.pallas import tpu_sc as plsc

N = 10000
E = 320000
NUM_FEAT = 128
HID = 64
EXP = 128
L = 12
EPS = 1e-7

NC = 2
NS = 16
NW = NC * NS
EPW = E // NW
CH = 80
NCHUNK = EPW // CH
NP = N
RPW = NP // NS

PERM = np.concatenate([np.arange(0, 16), np.arange(32, 48),
                       np.arange(16, 32), np.arange(48, 64)])
PERM_MAT = np.zeros((HID, HID), np.float32)
PERM_MAT[np.arange(HID), PERM] = 1.0


def _sc_agg_body(ztab_ref, ea_ref, ei_ref, t_ref, zero_ref,
                 out_ref, src_all, dst_all, h0, h1, e0, e1, ob0, ob1, tv, acc,
                 sem0, sem1, ssem0, ssem1):
    cid = lax.axis_index("c")
    sid = lax.axis_index("s")
    wid = cid * NS + sid

    pltpu.sync_copy(t_ref, tv)
    pltpu.sync_copy(ei_ref.at[0, wid], src_all)
    pltpu.sync_copy(ei_ref.at[1, wid], dst_all)
    pltpu.sync_copy(zero_ref, acc.at[pl.ds(sid * RPW, RPW)])
    plsc.subcore_barrier()

    tval = tv[...]
    hbufs = (h0, h1)
    ebufs = (e0, e1)
    obufs = (ob0, ob1)
    sems = (sem0, sem1)
    ssems = (ssem0, ssem1)
    himask = jnp.full((16,), -65536, jnp.int32)

    def issue(c, b):
        pltpu.async_copy(ztab_ref.at[src_all.at[c]], hbufs[b], sems[b])
        pltpu.async_copy(ea_ref.at[pl.ds(wid * EPW + c * CH, CH)],
                         ebufs[b], sems[b])

    def scatter_wait(b):
        pltpu.make_async_copy(obufs[b], acc.at[pl.ds(0, CH)], ssems[b]).wait()

    def process(c, b):
        pltpu.make_async_copy(ztab_ref.at[pl.ds(0, CH)], hbufs[b], sems[b]).wait()
        pltpu.make_async_copy(ea_ref.at[pl.ds(0, CH)], ebufs[b], sems[b]).wait()
        pl.when(c >= 2)(lambda: scatter_wait(b))
        hb = hbufs[b]
        eb = ebufs[b]
        ob = obufs[b]

        @plsc.parallel_loop(0, CH, unroll=8)
        def _(r):
            for g in range(2):
                cc = g * 16
                hi32 = hb[r, pl.ds(cc, 16)]
                ei32 = eb[r, pl.ds(cc, 16)]
                h_lo = plsc.bitcast(jnp.left_shift(hi32, 16), jnp.float32)
                h_hi = plsc.bitcast(jnp.bitwise_and(hi32, himask), jnp.float32)
                e_lo = plsc.bitcast(jnp.left_shift(ei32, 16), jnp.float32)
                e_hi = plsc.bitcast(jnp.bitwise_and(ei32, himask), jnp.float32)
                for half, hv, ev in ((0, h_lo, e_lo), (1, h_hi, e_hi)):
                    col = g * 32 + half * 16
                    m = jnp.maximum(hv + ev, 0.0) + EPS
                    ex = jnp.exp(tval * m)
                    ob[r, pl.ds(col, 16)] = ex
                    ob[r, pl.ds(64 + col, 16)] = ex * m

        pltpu.async_copy(ob, acc.at[dst_all.at[c]], ssems[b], add=True)

    issue(0, 0)

    def it_body(it, _):
        c0 = it * 2
        issue(c0 + 1, 1)
        process(c0, 0)
        issue(c0 + 2, 0)
        process(c0 + 1, 1)
        return 0

    lax.fori_loop(0, (NCHUNK - 1) // 2, it_body, 0)
    process(NCHUNK - 1, 0)
    scatter_wait(1)
    scatter_wait(0)

    plsc.subcore_barrier()
    pltpu.sync_copy(acc.at[pl.ds(sid * RPW, RPW)],
                    out_ref.at[pl.ds(cid * NP + sid * RPW, RPW)])


def _sc_aggregate(ztab, ea, ei, tvec, zero_block):
    kern = pl.kernel(
        _sc_agg_body,
        out_type=jax.ShapeDtypeStruct((NC * NP, 128), jnp.float32),
        mesh=plsc.VectorSubcoreMesh(core_axis_name="c", subcore_axis_name="s",
                                    num_cores=NC, num_subcores=NS),
        scratch_types=[
            pltpu.VMEM((NCHUNK, CH), jnp.int32),
            pltpu.VMEM((NCHUNK, CH), jnp.int32),
            pltpu.VMEM((CH, HID // 2), jnp.int32),
            pltpu.VMEM((CH, HID // 2), jnp.int32),
            pltpu.VMEM((CH, HID // 2), jnp.int32),
            pltpu.VMEM((CH, HID // 2), jnp.int32),
            pltpu.VMEM((CH, 128), jnp.float32),
            pltpu.VMEM((CH, 128), jnp.float32),
            pltpu.VMEM((16,), jnp.float32),
            pltpu.VMEM_SHARED((NP, 128), jnp.float32),
            pltpu.SemaphoreType.DMA,
            pltpu.SemaphoreType.DMA,
            pltpu.SemaphoreType.DMA,
            pltpu.SemaphoreType.DMA,
        ],
        compiler_params=pltpu.CompilerParams(use_tc_tiling_on_sc=False,
                                             needs_layout_passes=False),
    )
    return kern(ztab, ea, ei, tvec, zero_block)


BN = 1000
NB = N // BN
EDGE_DIM = 4


def _ln(u, g, b):
    mu = jnp.mean(u, axis=-1, keepdims=True)
    var = jnp.mean((u - mu) ** 2, axis=-1, keepdims=True)
    return (u - mu) / jnp.sqrt(var + 1e-5) * g + b


def _pack2(z):
    lo = lax.bitcast_convert_type(z[:, :HID // 2].astype(jnp.bfloat16),
                                  jnp.uint16).astype(jnp.uint32)
    hi = lax.bitcast_convert_type(z[:, HID // 2:].astype(jnp.bfloat16),
                                  jnp.uint16).astype(jnp.uint32)
    return lax.bitcast_convert_type(lo | (hi << 16), jnp.int32)


def _node_encode_body(x_ref, w_ref, b_ref, o_ref, oz_ref):
    h = jnp.dot(x_ref[...], w_ref[...],
                preferred_element_type=jnp.float32) + b_ref[...]
    o_ref[...] = h
    oz_ref[...] = _pack2(h)


def _node_encode(x, w, b):
    return pl.pallas_call(
        _node_encode_body,
        out_shape=(jax.ShapeDtypeStruct((N, HID), jnp.float32),
                   jax.ShapeDtypeStruct((N, HID // 2), jnp.int32)),
        grid=(NB,),
        in_specs=[
            pl.BlockSpec((BN, NUM_FEAT), lambda i: (i, 0)),
            pl.BlockSpec((NUM_FEAT, HID), lambda i: (0, 0)),
            pl.BlockSpec((1, HID), lambda i: (0, 0)),
        ],
        out_specs=(pl.BlockSpec((BN, HID), lambda i: (i, 0)),
                   pl.BlockSpec((BN, HID // 2), lambda i: (i, 0))),
    )(x, w, b)


BE = 16000
NEB = E // BE


def _edge_encode_body(a_ref, w_ref, b_ref, o_ref):
    a = a_ref[...]
    w = w_ref[...]
    acc = jnp.broadcast_to(b_ref[...], (BE, HID))
    for k in range(EDGE_DIM):
        acc = acc + a[:, k:k + 1] * w[k:k + 1, :]
    o_ref[...] = _pack2(acc)


def _edge_encode(ea, w, b):
    return pl.pallas_call(
        _edge_encode_body,
        out_shape=jax.ShapeDtypeStruct((E, HID // 2), jnp.int32),
        grid=(NEB,),
        in_specs=[
            pl.BlockSpec((BE, EDGE_DIM), lambda i: (i, 0)),
            pl.BlockSpec((EDGE_DIM, HID), lambda i: (0, 0)),
            pl.BlockSpec((1, HID), lambda i: (0, 0)),
        ],
        out_specs=pl.BlockSpec((BE, HID // 2), lambda i: (i, 0)),
    )(ea, w, b)


def _make_layer_body(first, last):
    def body(*refs):
        if last:
            (p_ref, pm_ref, z_ref, h_ref, w1_ref, b1_ref, lng_ref, lnb_ref,
             w2_ref, b2_ref, g2_ref, bb2_ref, lw_ref, lb_ref, oh_ref) = refs
        else:
            (p_ref, pm_ref, z_ref, h_ref, w1_ref, b1_ref, lng_ref, lnb_ref,
             w2_ref, b2_ref, g2_ref, bb2_ref, oh_ref, oz_ref, ozb_ref) = refs
        p = p_ref[...]
        acc = p[0] + p[1]
        den = acc[:, :HID]
        num = acc[:, HID:]
        agg = num / (den + 1e-16)
        agg = jnp.dot(agg, pm_ref[...], preferred_element_type=jnp.float32)
        out = agg + z_ref[...]
        u = jnp.dot(out, w1_ref[...], preferred_element_type=jnp.float32)
        u = jnp.maximum(_ln(u + b1_ref[...], lng_ref[...], lnb_ref[...]), 0.0)
        v = jnp.dot(u, w2_ref[...], preferred_element_type=jnp.float32)
        v = v + b2_ref[...]
        h_new = v if first else h_ref[...] + v
        zn = jnp.maximum(_ln(h_new, g2_ref[...], bb2_ref[...]), 0.0)
        if last:
            oh_ref[...] = jnp.dot(zn, lw_ref[...],
                                  preferred_element_type=jnp.float32) + lb_ref[...]
        else:
            oh_ref[...] = h_new
            oz_ref[...] = zn
            ozb_ref[...] = _pack2(zn)
    return body


def _layer_tc(partials, pm, z, h, w1, b1, lng, lnb, w2, b2, g2, bb2,
              first=False, last=False, lw=None, lb=None):
    p3 = partials.reshape(NC, NP, 128)
    node_spec = pl.BlockSpec((BN, HID), lambda i: (i, 0))
    small = lambda r, c: pl.BlockSpec((r, c), lambda i: (0, 0))
    in_specs = [
        pl.BlockSpec((NC, BN, 128), lambda i: (0, i, 0)),
        small(HID, HID),
        node_spec,
        node_spec,
        small(HID, EXP),
        small(1, EXP),
        small(1, EXP),
        small(1, EXP),
        small(EXP, HID),
        small(1, HID),
        small(1, HID),
        small(1, HID),
    ]
    args = [p3, pm, z, h, w1, b1, lng, lnb, w2, b2, g2, bb2]
    if last:
        in_specs += [small(HID, HID), small(1, HID)]
        args += [lw, lb]
        out_shape = jax.ShapeDtypeStruct((N, HID), jnp.float32)
        out_specs = node_spec
    else:
        out_shape = (jax.ShapeDtypeStruct((N, HID), jnp.float32),
                     jax.ShapeDtypeStruct((N, HID), jnp.float32),
                     jax.ShapeDtypeStruct((N, HID // 2), jnp.int32))
        out_specs = (node_spec, node_spec,
                     pl.BlockSpec((BN, HID // 2), lambda i: (i, 0)))
    return pl.pallas_call(
        _make_layer_body(first, last),
        out_shape=out_shape,
        grid=(NB,),
        in_specs=in_specs,
        out_specs=out_specs,
    )(*args)


def kernel(x, edge_attr, node_W, node_b, edge_W, edge_b, t, conv_W1, conv_b1,
           conv_ln_g, conv_ln_b, conv_W2, conv_b2, layer_ln_g, layer_ln_b,
           lin_W, lin_b, edge_index):
    ei = edge_index.reshape(2, NW, NCHUNK, CH)
    zero_block = jnp.zeros((RPW, 128), jnp.float32)
    pm = jnp.asarray(PERM_MAT)

    h0, z0b = _node_encode(x, node_W, node_b.reshape(1, HID))
    ea = _edge_encode(edge_attr, edge_W, edge_b.reshape(1, HID))

    def r2(a):
        return a.reshape(1, -1)

    h = None
    z = h0
    zb = z0b
    for i in range(L):
        tvec = jnp.broadcast_to(t[i], (16,)).astype(jnp.float32)
        partials = _sc_aggregate(zb, ea, ei, tvec, zero_block)
        first = (i == 0)
        last = (i == L - 1)
        g2 = layer_ln_g[0] if last else layer_ln_g[i + 1]
        bb2 = layer_ln_b[0] if last else layer_ln_b[i + 1]
        res = _layer_tc(
            partials, pm, z, (z if first else h),
            conv_W1[i], r2(conv_b1[i]), r2(conv_ln_g[i]), r2(conv_ln_b[i]),
            conv_W2[i], r2(conv_b2[i]), r2(g2), r2(bb2),
            first=first, last=last,
            lw=(lin_W if last else None),
            lb=(r2(lin_b) if last else None),
        )
        if last:
            return res
        h, z, zb = res


# speedup vs baseline: 1.0988x; 1.0988x over previous
import numpy as np

import jax
import jax.numpy as jnp
from jax import lax
from jax.experimental import pallas as pl
from jax.experimental.pallas import tpu as pltpu
from jax.experimental.pallas import tpu_sc as plsc

N = 10000
E = 320000
NUM_FEAT = 128
HID = 64
EXP = 128
L = 12
EPS = 1e-7

NC = 2
NS = 16
NW = NC * NS
EPW = E // NW
CH = 80
NCHUNK = EPW // CH
NP = N
RPW = NP // NS

PERM = np.concatenate([np.arange(0, 16), np.arange(32, 48),
                       np.arange(16, 32), np.arange(48, 64)])
PERM_MAT = np.zeros((HID, HID), np.float32)
PERM_MAT[np.arange(HID), PERM] = 1.0


def _sc_agg_body(ztab_ref, ea_ref, ei_ref, t_ref, zero_ref,
                 out_ref, src_all, dst_all, h0, h1, e0, e1, ob0, ob1, tv, acc,
                 sem0, sem1, ssem0, ssem1):
    cid = lax.axis_index("c")
    sid = lax.axis_index("s")
    wid = cid * NS + sid

    pltpu.sync_copy(t_ref, tv)
    pltpu.sync_copy(ei_ref.at[0, wid], src_all)
    pltpu.sync_copy(ei_ref.at[1, wid], dst_all)
    pltpu.sync_copy(zero_ref, acc.at[pl.ds(sid * RPW, RPW)])
    plsc.subcore_barrier()

    tval = tv[...]
    hbufs = (h0, h1)
    ebufs = (e0, e1)
    obufs = (ob0, ob1)
    sems = (sem0, sem1)
    ssems = (ssem0, ssem1)
    himask = jnp.full((16,), -65536, jnp.int32)

    def issue(c, b):
        pltpu.async_copy(ztab_ref.at[src_all.at[c]], hbufs[b], sems[b])
        pltpu.async_copy(ea_ref.at[pl.ds(wid * EPW + c * CH, CH)],
                         ebufs[b], sems[b])

    def scatter_wait(b):
        pltpu.make_async_copy(obufs[b], acc.at[pl.ds(0, CH)], ssems[b]).wait()

    def process(c, b):
        pltpu.make_async_copy(ztab_ref.at[pl.ds(0, CH)], hbufs[b], sems[b]).wait()
        pltpu.make_async_copy(ea_ref.at[pl.ds(0, CH)], ebufs[b], sems[b]).wait()
        pl.when(c >= 2)(lambda: scatter_wait(b))
        hb = hbufs[b]
        eb = ebufs[b]
        ob = obufs[b]

        @plsc.parallel_loop(0, CH, unroll=4)
        def _(r):
            for g in range(2):
                cc = g * 16
                hi32 = hb[r, pl.ds(cc, 16)]
                ei32 = eb[r, pl.ds(cc, 16)]
                h_lo = plsc.bitcast(jnp.left_shift(hi32, 16), jnp.float32)
                h_hi = plsc.bitcast(jnp.bitwise_and(hi32, himask), jnp.float32)
                e_lo = plsc.bitcast(jnp.left_shift(ei32, 16), jnp.float32)
                e_hi = plsc.bitcast(jnp.bitwise_and(ei32, himask), jnp.float32)
                for half, hv, ev in ((0, h_lo, e_lo), (1, h_hi, e_hi)):
                    col = g * 32 + half * 16
                    m = jnp.maximum(hv + ev, 0.0) + EPS
                    ex = jnp.exp(tval * m)
                    ob[r, pl.ds(col, 16)] = ex
                    ob[r, pl.ds(64 + col, 16)] = ex * m

        pltpu.async_copy(ob, acc.at[dst_all.at[c]], ssems[b], add=True)

    issue(0, 0)

    def it_body(it, _):
        c0 = it * 2
        issue(c0 + 1, 1)
        process(c0, 0)
        issue(c0 + 2, 0)
        process(c0 + 1, 1)
        return 0

    lax.fori_loop(0, (NCHUNK - 1) // 2, it_body, 0)
    process(NCHUNK - 1, 0)
    scatter_wait(1)
    scatter_wait(0)

    plsc.subcore_barrier()
    pltpu.sync_copy(acc.at[pl.ds(sid * RPW, RPW)],
                    out_ref.at[pl.ds(cid * NP + sid * RPW, RPW)])


def _sc_aggregate(ztab, ea, ei, tvec, zero_block):
    kern = pl.kernel(
        _sc_agg_body,
        out_type=jax.ShapeDtypeStruct((NC * NP, 128), jnp.float32),
        mesh=plsc.VectorSubcoreMesh(core_axis_name="c", subcore_axis_name="s",
                                    num_cores=NC, num_subcores=NS),
        scratch_types=[
            pltpu.VMEM((NCHUNK, CH), jnp.int32),
            pltpu.VMEM((NCHUNK, CH), jnp.int32),
            pltpu.VMEM((CH, HID // 2), jnp.int32),
            pltpu.VMEM((CH, HID // 2), jnp.int32),
            pltpu.VMEM((CH, HID // 2), jnp.int32),
            pltpu.VMEM((CH, HID // 2), jnp.int32),
            pltpu.VMEM((CH, 128), jnp.float32),
            pltpu.VMEM((CH, 128), jnp.float32),
            pltpu.VMEM((16,), jnp.float32),
            pltpu.VMEM_SHARED((NP, 128), jnp.float32),
            pltpu.SemaphoreType.DMA,
            pltpu.SemaphoreType.DMA,
            pltpu.SemaphoreType.DMA,
            pltpu.SemaphoreType.DMA,
        ],
        compiler_params=pltpu.CompilerParams(use_tc_tiling_on_sc=False,
                                             needs_layout_passes=False),
    )
    return kern(ztab, ea, ei, tvec, zero_block)


BN = 1000
NB = N // BN
EDGE_DIM = 4


def _ln(u, g, b):
    mu = jnp.mean(u, axis=-1, keepdims=True)
    var = jnp.mean((u - mu) ** 2, axis=-1, keepdims=True)
    return (u - mu) / jnp.sqrt(var + 1e-5) * g + b


def _pack2(z):
    lo = lax.bitcast_convert_type(z[:, :HID // 2].astype(jnp.bfloat16),
                                  jnp.uint16).astype(jnp.uint32)
    hi = lax.bitcast_convert_type(z[:, HID // 2:].astype(jnp.bfloat16),
                                  jnp.uint16).astype(jnp.uint32)
    return lax.bitcast_convert_type(lo | (hi << 16), jnp.int32)


def _node_encode_body(x_ref, w_ref, b_ref, o_ref, oz_ref):
    h = jnp.dot(x_ref[...], w_ref[...],
                preferred_element_type=jnp.float32) + b_ref[...]
    o_ref[...] = h
    oz_ref[...] = _pack2(h)


def _node_encode(x, w, b):
    return pl.pallas_call(
        _node_encode_body,
        out_shape=(jax.ShapeDtypeStruct((N, HID), jnp.float32),
                   jax.ShapeDtypeStruct((N, HID // 2), jnp.int32)),
        grid=(NB,),
        in_specs=[
            pl.BlockSpec((BN, NUM_FEAT), lambda i: (i, 0)),
            pl.BlockSpec((NUM_FEAT, HID), lambda i: (0, 0)),
            pl.BlockSpec((1, HID), lambda i: (0, 0)),
        ],
        out_specs=(pl.BlockSpec((BN, HID), lambda i: (i, 0)),
                   pl.BlockSpec((BN, HID // 2), lambda i: (i, 0))),
    )(x, w, b)


BE = 16000
NEB = E // BE


def _edge_encode_body(a_ref, w_ref, b_ref, o_ref):
    a = a_ref[...]
    w = w_ref[...]
    acc = jnp.dot(a, w, preferred_element_type=jnp.float32) + b_ref[...]
    o_ref[...] = _pack2(acc)


def _edge_encode(ea, w, b):
    return pl.pallas_call(
        _edge_encode_body,
        out_shape=jax.ShapeDtypeStruct((E, HID // 2), jnp.int32),
        grid=(NEB,),
        in_specs=[
            pl.BlockSpec((BE, EDGE_DIM), lambda i: (i, 0)),
            pl.BlockSpec((EDGE_DIM, HID), lambda i: (0, 0)),
            pl.BlockSpec((1, HID), lambda i: (0, 0)),
        ],
        out_specs=pl.BlockSpec((BE, HID // 2), lambda i: (i, 0)),
    )(ea, w, b)


def _make_layer_body(first, last):
    def body(*refs):
        if last:
            (p_ref, pm_ref, z_ref, h_ref, w1_ref, b1_ref, lng_ref, lnb_ref,
             w2_ref, b2_ref, g2_ref, bb2_ref, lw_ref, lb_ref, oh_ref) = refs
        else:
            (p_ref, pm_ref, z_ref, h_ref, w1_ref, b1_ref, lng_ref, lnb_ref,
             w2_ref, b2_ref, g2_ref, bb2_ref, oh_ref, oz_ref, ozb_ref) = refs
        p = p_ref[...]
        acc = p[0] + p[1]
        den = acc[:, :HID]
        num = acc[:, HID:]
        agg = num / (den + 1e-16)
        agg = jnp.dot(agg, pm_ref[...], preferred_element_type=jnp.float32)
        out = agg + z_ref[...]
        u = jnp.dot(out, w1_ref[...], preferred_element_type=jnp.float32)
        u = jnp.maximum(_ln(u + b1_ref[...], lng_ref[...], lnb_ref[...]), 0.0)
        v = jnp.dot(u, w2_ref[...], preferred_element_type=jnp.float32)
        v = v + b2_ref[...]
        h_new = v if first else h_ref[...] + v
        zn = jnp.maximum(_ln(h_new, g2_ref[...], bb2_ref[...]), 0.0)
        if last:
            oh_ref[...] = jnp.dot(zn, lw_ref[...],
                                  preferred_element_type=jnp.float32) + lb_ref[...]
        else:
            oh_ref[...] = h_new
            oz_ref[...] = zn
            ozb_ref[...] = _pack2(zn)
    return body


def _layer_tc(partials, pm, z, h, w1, b1, lng, lnb, w2, b2, g2, bb2,
              first=False, last=False, lw=None, lb=None):
    p3 = partials.reshape(NC, NP, 128)
    node_spec = pl.BlockSpec((BN, HID), lambda i: (i, 0))
    small = lambda r, c: pl.BlockSpec((r, c), lambda i: (0, 0))
    in_specs = [
        pl.BlockSpec((NC, BN, 128), lambda i: (0, i, 0)),
        small(HID, HID),
        node_spec,
        node_spec,
        small(HID, EXP),
        small(1, EXP),
        small(1, EXP),
        small(1, EXP),
        small(EXP, HID),
        small(1, HID),
        small(1, HID),
        small(1, HID),
    ]
    args = [p3, pm, z, h, w1, b1, lng, lnb, w2, b2, g2, bb2]
    if last:
        in_specs += [small(HID, HID), small(1, HID)]
        args += [lw, lb]
        out_shape = jax.ShapeDtypeStruct((N, HID), jnp.float32)
        out_specs = node_spec
    else:
        out_shape = (jax.ShapeDtypeStruct((N, HID), jnp.float32),
                     jax.ShapeDtypeStruct((N, HID), jnp.float32),
                     jax.ShapeDtypeStruct((N, HID // 2), jnp.int32))
        out_specs = (node_spec, node_spec,
                     pl.BlockSpec((BN, HID // 2), lambda i: (i, 0)))
    return pl.pallas_call(
        _make_layer_body(first, last),
        out_shape=out_shape,
        grid=(NB,),
        in_specs=in_specs,
        out_specs=out_specs,
    )(*args)


def kernel(x, edge_attr, node_W, node_b, edge_W, edge_b, t, conv_W1, conv_b1,
           conv_ln_g, conv_ln_b, conv_W2, conv_b2, layer_ln_g, layer_ln_b,
           lin_W, lin_b, edge_index):
    ei = edge_index.reshape(2, NW, NCHUNK, CH)
    zero_block = jnp.zeros((RPW, 128), jnp.float32)
    pm = jnp.asarray(PERM_MAT)

    h0, z0b = _node_encode(x, node_W, node_b.reshape(1, HID))
    ea = _edge_encode(edge_attr, edge_W, edge_b.reshape(1, HID))

    def r2(a):
        return a.reshape(1, -1)

    h = None
    z = h0
    zb = z0b
    for i in range(L):
        tvec = jnp.broadcast_to(t[i], (16,)).astype(jnp.float32)
        partials = _sc_aggregate(zb, ea, ei, tvec, zero_block)
        first = (i == 0)
        last = (i == L - 1)
        g2 = layer_ln_g[0] if last else layer_ln_g[i + 1]
        bb2 = layer_ln_b[0] if last else layer_ln_b[i + 1]
        res = _layer_tc(
            partials, pm, z, (z if first else h),
            conv_W1[i], r2(conv_b1[i]), r2(conv_ln_g[i]), r2(conv_ln_b[i]),
            conv_W2[i], r2(conv_b2[i]), r2(g2), r2(bb2),
            first=first, last=last,
            lw=(lin_W if last else None),
            lb=(r2(lin_b) if last else None),
        )
        if last:
            return res
        h, z, zb = res
